# baseline (speedup 1.0000x reference)
import functools

import jax
import jax.numpy as jnp
from jax import lax
from jax.experimental import pallas as pl
from jax.experimental.pallas import tpu as pltpu
from jax.experimental.pallas import tpu_sc as plsc

N = 10000
D = 128
E = 320000
NC = 2
NS = 16
NW = NC * NS
EPW = E // NW
CH = 80
NCHUNK = EPW // CH
NG = 5
GC = NCHUNK // NG
ACC_N = 10240
ROWS_PT = ACC_N // NS
LANES = 16

_mesh = plsc.VectorSubcoreMesh(core_axis_name="c", subcore_axis_name="s")


def _sc_agg_body(h_hbm, src_hbm, dst_hbm, w_hbm, out_hbm,
                 src_v, dst_v, w_v, rows_a, rows_b, acc_sh, sem_g, sem_s):
    cid = lax.axis_index("c")
    sid = lax.axis_index("s")
    wid = sid * NC + cid
    bufs = (rows_a, rows_b)

    zero16 = jnp.zeros((LANES,), jnp.float32)
    for r in range(CH):
        for j in range(D // LANES):
            rows_a[r, pl.ds(j * LANES, LANES)] = zero16
    rb = sid * ROWS_PT
    nfull = ROWS_PT // CH
    for t in range(nfull):
        pltpu.sync_copy(rows_a, acc_sh.at[pl.ds(rb + t * CH, CH)])
    rem = ROWS_PT - nfull * CH
    if rem:
        pltpu.sync_copy(rows_a.at[pl.ds(0, rem)],
                        acc_sh.at[pl.ds(rb + nfull * CH, rem)])
    plsc.subcore_barrier()

    def scale(rows_v, c):
        for i in range(CH):
            if i % LANES == 0:
                wvec = w_v[c, pl.ds(i, LANES)]
            w = wvec[i % LANES]
            for j in range(D // LANES):
                sl = pl.ds(j * LANES, LANES)
                rows_v[i, sl] = rows_v[i, sl] * w

    def gather(buf, c):
        pltpu.make_async_copy(h_hbm.at[src_v.at[c]], buf, sem_g).start()

    def wait_gather(buf, c):
        pltpu.make_async_copy(h_hbm.at[src_v.at[c]], buf, sem_g).wait()

    def scatter(buf, c):
        pltpu.make_async_copy(buf, acc_sh.at[dst_v.at[c]], sem_s).start(
            add=True)

    def wait_scatter(buf, c):
        pltpu.make_async_copy(buf, acc_sh.at[dst_v.at[c]], sem_s).wait()

    def group(g, carry):
        pltpu.sync_copy(src_hbm.at[wid, g], src_v)
        pltpu.sync_copy(dst_hbm.at[wid, g], dst_v)
        pltpu.sync_copy(w_hbm.at[wid, g], w_v)

        gather(rows_a, 0)

        def pair(k, carry2):
            ce = 2 * k
            co = ce + 1
            wait_gather(rows_a, ce)

            @pl.when(k > 0)
            def _():
                wait_scatter(rows_b, ce - 1)

            gather(rows_b, co)
            scale(rows_a, ce)
            scatter(rows_a, ce)
            wait_gather(rows_b, co)
            scale(rows_b, co)
            scatter(rows_b, co)
            wait_scatter(rows_a, ce)
            gather(rows_a, co + 1)
            return carry2

        lax.fori_loop(0, GC // 2, pair, 0)
        wait_gather(rows_a, GC - 1)
        scale(rows_a, GC - 1)
        wait_scatter(rows_b, GC - 2)
        scatter(rows_a, GC - 1)
        wait_scatter(rows_a, GC - 1)
        return carry

    lax.fori_loop(0, NG, group, 0)

    plsc.subcore_barrier()
    pltpu.sync_copy(acc_sh.at[pl.ds(rb, ROWS_PT)],
                    out_hbm.at[cid, pl.ds(rb, ROWS_PT)])


_sc_aggregate = functools.partial(
    pl.kernel,
    out_type=jax.ShapeDtypeStruct((NC, ACC_N, D), jnp.float32),
    mesh=_mesh,
    scratch_types=[
        pltpu.VMEM((GC, CH), jnp.int32),
        pltpu.VMEM((GC, CH), jnp.int32),
        pltpu.VMEM((GC, CH), jnp.float32),
        pltpu.VMEM((CH, D), jnp.float32),
        pltpu.VMEM((CH, D), jnp.float32),
        pltpu.VMEM_SHARED((ACC_N, D), jnp.float32),
        pltpu.SemaphoreType.DMA,
        pltpu.SemaphoreType.DMA,
    ],
)(_sc_agg_body)


BLK = 400
GRID = N // BLK


def _mm1_body(x_ref, w_ref, o_ref):
    o_ref[...] = jnp.dot(x_ref[...], w_ref[...],
                         preferred_element_type=jnp.float32)


def _mm2_body(p_ref, b_ref, w_ref, o_ref):
    s = p_ref[0] + p_ref[1] + b_ref[0]
    o_ref[...] = jnp.dot(jnp.maximum(s, 0.0), w_ref[...],
                         preferred_element_type=jnp.float32)


def _final_body(p_ref, b_ref, o_ref):
    o_ref[...] = p_ref[0] + p_ref[1] + b_ref[0]


def _matmul(x, W):
    return pl.pallas_call(
        _mm1_body,
        grid=(GRID,),
        in_specs=[pl.BlockSpec((BLK, D), lambda i: (i, 0)),
                  pl.BlockSpec((D, D), lambda i: (0, 0))],
        out_specs=pl.BlockSpec((BLK, D), lambda i: (i, 0)),
        out_shape=jax.ShapeDtypeStruct((N, D), jnp.float32),
    )(x, W)


def _fused_mm2(p, b, W):
    return pl.pallas_call(
        _mm2_body,
        grid=(GRID,),
        in_specs=[pl.BlockSpec((NC, BLK, D), lambda i: (0, i, 0)),
                  pl.BlockSpec((1, D), lambda i: (0, 0)),
                  pl.BlockSpec((D, D), lambda i: (0, 0))],
        out_specs=pl.BlockSpec((BLK, D), lambda i: (i, 0)),
        out_shape=jax.ShapeDtypeStruct((N, D), jnp.float32),
    )(p, b.reshape(1, D), W)


def _final(p, b):
    return pl.pallas_call(
        _final_body,
        grid=(GRID,),
        in_specs=[pl.BlockSpec((NC, BLK, D), lambda i: (0, i, 0)),
                  pl.BlockSpec((1, D), lambda i: (0, 0))],
        out_specs=pl.BlockSpec((BLK, D), lambda i: (i, 0)),
        out_shape=jax.ShapeDtypeStruct((N, D), jnp.float32),
    )(p, b.reshape(1, D))


def kernel(x, edge_index, edge_weight, W1, b1, W2, b2):
    src = edge_index[0].reshape(NW, NG, GC, CH)
    dst = edge_index[1].reshape(NW, NG, GC, CH)
    w = edge_weight.reshape(NW, NG, GC, CH)

    h1 = _matmul(x, W1)
    p1 = _sc_aggregate(h1, src, dst, w)
    h2 = _fused_mm2(p1, b1, W2)
    p2 = _sc_aggregate(h2, src, dst, w)
    return _final(p2, b2)


# speedup vs baseline: 1.5064x; 1.5064x over previous
import functools

import jax
import jax.numpy as jnp
from jax import lax
from jax.experimental import pallas as pl
from jax.experimental.pallas import tpu as pltpu
from jax.experimental.pallas import tpu_sc as plsc

N = 10000
D = 128
E = 320000
NC = 2
NS = 16
NW = NC * NS
EPW = E // NW
CH = 80
NCHUNK = EPW // CH
NG = 5
GC = NCHUNK // NG
ACC_N = 10240
ROWS_PT = ACC_N // NS
LANES = 16

_mesh = plsc.VectorSubcoreMesh(core_axis_name="c", subcore_axis_name="s")


def _sc_agg_body(h_hbm, src_hbm, dst_hbm, w_hbm, out_hbm,
                 src_v, dst_v, w_v, rows_a, rows_b, rows_c,
                 acc_sh, sem_g, sem_s):
    cid = lax.axis_index("c")
    sid = lax.axis_index("s")
    wid = sid * NC + cid

    zero16 = jnp.zeros((LANES,), jnp.float32)

    def zrow(r, carry):
        for j in range(D // LANES):
            rows_a[r, pl.ds(j * LANES, LANES)] = zero16
        return carry

    lax.fori_loop(0, CH, zrow, 0)
    rb = sid * ROWS_PT
    for t in range(ROWS_PT // CH):
        pltpu.sync_copy(rows_a, acc_sh.at[pl.ds(rb + t * CH, CH)])
    plsc.subcore_barrier()

    def scale(rows_v, c):
        def grp(i, carry):
            base = i * LANES
            wvec = w_v[c, pl.ds(base, LANES)]
            for l in range(LANES):
                w = wvec[l]
                for j in range(D // LANES):
                    sl = pl.ds(j * LANES, LANES)
                    rows_v[base + l, sl] = rows_v[base + l, sl] * w
            return carry

        lax.fori_loop(0, CH // LANES, grp, 0)

    def gather(buf, c):
        pltpu.make_async_copy(h_hbm.at[src_v.at[c]], buf, sem_g).start()

    def wait_gather(buf, c):
        pltpu.make_async_copy(h_hbm.at[src_v.at[c]], buf, sem_g).wait()

    def scatter(buf, c):
        pltpu.make_async_copy(buf, acc_sh.at[dst_v.at[c]], sem_s).start(
            add=True)

    def wait_scatter(buf, c):
        pltpu.make_async_copy(buf, acc_sh.at[dst_v.at[c]], sem_s).wait()

    bufs = (rows_a, rows_b, rows_c)

    def group(g, carry):
        pltpu.sync_copy(src_hbm.at[wid, g], src_v)
        pltpu.sync_copy(dst_hbm.at[wid, g], dst_v)
        pltpu.sync_copy(w_hbm.at[wid, g], w_v)

        gather(rows_a, 0)
        gather(rows_b, 1)
        gather(rows_c, 2)

        def triad(k, carry2):
            for j in range(3):
                c = 3 * k + j
                buf = bufs[j]
                wait_gather(buf, c)
                scale(buf, c)

                @pl.when(k > 0)
                def _():
                    wait_scatter(buf, c - 3)

                @pl.when(c + 3 < GC)
                def _():
                    gather(buf, c + 3)

                scatter(buf, c)
            return carry2

        lax.fori_loop(0, GC // 3, triad, 0)
        wait_gather(rows_a, GC - 1)
        scale(rows_a, GC - 1)
        wait_scatter(rows_a, GC - 4)
        scatter(rows_a, GC - 1)
        wait_scatter(rows_b, GC - 3)
        wait_scatter(rows_c, GC - 2)
        wait_scatter(rows_a, GC - 1)
        return carry

    lax.fori_loop(0, NG, group, 0)

    plsc.subcore_barrier()
    pltpu.sync_copy(acc_sh.at[pl.ds(rb, ROWS_PT)],
                    out_hbm.at[cid, pl.ds(rb, ROWS_PT)])


_sc_aggregate = functools.partial(
    pl.kernel,
    out_type=jax.ShapeDtypeStruct((NC, ACC_N, D), jnp.float32),
    mesh=_mesh,
    scratch_types=[
        pltpu.VMEM((GC, CH), jnp.int32),
        pltpu.VMEM((GC, CH), jnp.int32),
        pltpu.VMEM((GC, CH), jnp.float32),
        pltpu.VMEM((CH, D), jnp.float32),
        pltpu.VMEM((CH, D), jnp.float32),
        pltpu.VMEM((CH, D), jnp.float32),
        pltpu.VMEM_SHARED((ACC_N, D), jnp.float32),
        pltpu.SemaphoreType.DMA,
        pltpu.SemaphoreType.DMA,
    ],
)(_sc_agg_body)


BLK = 400
GRID = N // BLK


def _mm1_body(x_ref, w_ref, o_ref):
    o_ref[...] = jnp.dot(x_ref[...], w_ref[...],
                         preferred_element_type=jnp.float32)


def _mm2_body(p_ref, b_ref, w_ref, o_ref):
    s = p_ref[0] + p_ref[1] + b_ref[0]
    o_ref[...] = jnp.dot(jnp.maximum(s, 0.0), w_ref[...],
                         preferred_element_type=jnp.float32)


def _final_body(p_ref, b_ref, o_ref):
    o_ref[...] = p_ref[0] + p_ref[1] + b_ref[0]


def _matmul(x, W):
    return pl.pallas_call(
        _mm1_body,
        grid=(GRID,),
        in_specs=[pl.BlockSpec((BLK, D), lambda i: (i, 0)),
                  pl.BlockSpec((D, D), lambda i: (0, 0))],
        out_specs=pl.BlockSpec((BLK, D), lambda i: (i, 0)),
        out_shape=jax.ShapeDtypeStruct((N, D), jnp.float32),
    )(x, W)


def _fused_mm2(p, b, W):
    return pl.pallas_call(
        _mm2_body,
        grid=(GRID,),
        in_specs=[pl.BlockSpec((NC, BLK, D), lambda i: (0, i, 0)),
                  pl.BlockSpec((1, D), lambda i: (0, 0)),
                  pl.BlockSpec((D, D), lambda i: (0, 0))],
        out_specs=pl.BlockSpec((BLK, D), lambda i: (i, 0)),
        out_shape=jax.ShapeDtypeStruct((N, D), jnp.float32),
    )(p, b.reshape(1, D), W)


def _final(p, b):
    return pl.pallas_call(
        _final_body,
        grid=(GRID,),
        in_specs=[pl.BlockSpec((NC, BLK, D), lambda i: (0, i, 0)),
                  pl.BlockSpec((1, D), lambda i: (0, 0))],
        out_specs=pl.BlockSpec((BLK, D), lambda i: (i, 0)),
        out_shape=jax.ShapeDtypeStruct((N, D), jnp.float32),
    )(p, b.reshape(1, D))


def kernel(x, edge_index, edge_weight, W1, b1, W2, b2):
    src = edge_index[0].reshape(NW, NG, GC, CH)
    dst = edge_index[1].reshape(NW, NG, GC, CH)
    w = edge_weight.reshape(NW, NG, GC, CH)

    h1 = _matmul(x, W1)
    p1 = _sc_aggregate(h1, src, dst, w)
    h2 = _fused_mm2(p1, b1, W2)
    p2 = _sc_aggregate(h2, src, dst, w)
    return _final(p2, b2)
